# Initial kernel scaffold; baseline (speedup 1.0000x reference)
import jax
import jax.numpy as jnp
from jax.experimental import pallas as pl


def kernel(x, edge_index, bit_sum, W, lin_b, att, b_out):
    raise NotImplementedError("write your pallas kernel here")


# speedup vs baseline: 7.7142x; 7.7142x over previous
import dataclasses

import jax
import jax.numpy as jnp
from jax import lax
from jax.experimental import pallas as pl
from jax.experimental.pallas import tpu as pltpu
from jax.experimental.pallas import tpu_sc as plsc

N = 10000
NP = 10240
E = 320000
D = 128
NC = 2
NS = 16
NW = NC * NS
E2 = 327680
EPT = E2 // NW
SUBC = 64
NSC = EPT // SUBC
NPT = NP // NS
BLK = 1024
GRID = NP // BLK


def _pre_body(x_ref, w_ref, b_ref, ai_ref, aj_ref, fea_ref, s2_ref, mx_ref):
    i = pl.program_id(0)
    fea = jnp.dot(x_ref[...], w_ref[...], preferred_element_type=jnp.float32,
                  precision=lax.Precision.HIGHEST) + b_ref[...]
    fea_ref[...] = fea
    dn = (((1,), (1,)), ((), ()))
    si = lax.dot_general(ai_ref[...], fea, dn, preferred_element_type=jnp.float32,
                         precision=lax.Precision.HIGHEST)
    sj = lax.dot_general(aj_ref[...], fea, dn, preferred_element_type=jnp.float32,
                         precision=lax.Precision.HIGHEST)
    s2_ref[...] = jnp.concatenate(
        [si, sj, jnp.zeros((6, si.shape[1]), jnp.float32)], axis=0)

    @pl.when(i == 0)
    def _():
        mx_ref[...] = jnp.full((8, 128), -1e30, jnp.float32)

    rows = lax.broadcasted_iota(jnp.int32, (8, 128), 0)
    mx_ref[...] = jnp.maximum(mx_ref[...],
                              jnp.where(rows < 4, jnp.max(si), jnp.max(sj)))


def _sc_body(s2_hbm, eidx_hbm, fea_hbm, bv_hbm, den_hbm, agg_hbm,
             si_v, sj_v, den_v, rows_v, dst_b, src_b, bv_v, acc_s):
    cid = lax.axis_index("c")
    sid = lax.axis_index("s")
    wid = cid * NS + sid

    pltpu.sync_copy(s2_hbm.at[0], si_v)
    pltpu.sync_copy(s2_hbm.at[1], sj_v)
    pltpu.sync_copy(bv_hbm, bv_v)
    bv = bv_v[...]

    z16 = jnp.zeros((16,), jnp.float32)

    @pl.loop(0, SUBC)
    def _(r):
        @pl.loop(0, D, step=16)
        def _(c):
            rows_v[r, pl.ds(c, 16)] = z16

    @pl.loop(0, NPT, step=SUBC)
    def _(r):
        pltpu.sync_copy(rows_v, acc_s.at[pl.ds(sid * NPT + r, SUBC)])

    @pl.loop(0, NP, step=16)
    def _(i):
        den_v[pl.ds(i, 16)] = z16

    plsc.subcore_barrier()

    @pl.loop(0, NSC)
    def _(sc):
        off = sc * SUBC
        pltpu.sync_copy(eidx_hbm.at[1, wid, pl.ds(off, SUBC)], dst_b)
        pltpu.sync_copy(eidx_hbm.at[0, wid, pl.ds(off, SUBC)], src_b)
        pltpu.sync_copy(fea_hbm.at[src_b], rows_v)

        @pl.loop(0, SUBC, step=16)
        def _(g):
            d = dst_b[pl.ds(g, 16)]
            s = src_b[pl.ds(g, 16)]
            a = plsc.load_gather(si_v, [d]) + plsc.load_gather(sj_v, [s])
            a = jnp.where(a > 0, a, 0.2 * a)
            ex = jnp.exp(a - bv)
            plsc.addupdate_scatter(den_v, [d], ex)
            for j in range(16):
                e = ex[j]
                for c in range(0, D, 16):
                    rows_v[g + j, pl.ds(c, 16)] = rows_v[g + j, pl.ds(c, 16)] * e

        pltpu.sync_copy(rows_v, acc_s.at[dst_b], add=True)

    pltpu.sync_copy(den_v, den_hbm.at[wid])

    plsc.subcore_barrier()

    pltpu.sync_copy(acc_s.at[pl.ds(sid * NPT, NPT)],
                    agg_hbm.at[cid, pl.ds(sid * NPT, NPT)])


def _post_body(ap_ref, dp_ref, x_ref, bo_ref, o_ref):
    agg = ap_ref[0] + ap_ref[1]
    den = jnp.sum(dp_ref[...], axis=1, keepdims=True)
    o_ref[...] = agg / (den + 1e-16) + x_ref[...] + bo_ref[...]


def kernel(x, edge_index, bit_sum, W, lin_b, att, b_out):
    xp = jnp.concatenate([x, jnp.zeros((NP - N, D), jnp.float32)], axis=0)
    ai = att[0, 0, :D].reshape(1, D)
    aj = att[0, 0, D:].reshape(1, D)
    lb2 = lin_b.reshape(1, D)
    bo2 = b_out.reshape(1, D)
    epad = jnp.full((2, E2 - E), N, dtype=jnp.int32)
    eidx = jnp.concatenate([edge_index, epad], axis=1).reshape(2, NW, EPT)

    fea, s2, mx = pl.pallas_call(
        _pre_body,
        grid=(GRID,),
        in_specs=[
            pl.BlockSpec((BLK, D), lambda i: (i, 0)),
            pl.BlockSpec((D, D), lambda i: (0, 0)),
            pl.BlockSpec((1, D), lambda i: (0, 0)),
            pl.BlockSpec((1, D), lambda i: (0, 0)),
            pl.BlockSpec((1, D), lambda i: (0, 0)),
        ],
        out_specs=[
            pl.BlockSpec((BLK, D), lambda i: (i, 0)),
            pl.BlockSpec((8, BLK), lambda i: (0, i)),
            pl.BlockSpec((8, 128), lambda i: (0, 0)),
        ],
        out_shape=[
            jax.ShapeDtypeStruct((NP, D), jnp.float32),
            jax.ShapeDtypeStruct((8, NP), jnp.float32),
            jax.ShapeDtypeStruct((8, 128), jnp.float32),
        ],
    )(xp, W, lb2, ai, aj)

    b0 = mx[0, 0] + mx[4, 0]
    bound = jnp.where(b0 > 0, b0, 0.2 * b0)
    bvec = jnp.full((16,), bound, jnp.float32)

    mesh = plsc.VectorSubcoreMesh(core_axis_name="c", subcore_axis_name="s")
    cp = pltpu.CompilerParams()
    if "needs_layout_passes" in pltpu.CompilerParams.__dataclass_fields__:
        cp = dataclasses.replace(cp, needs_layout_passes=False)
    den_p, agg_p = pl.kernel(
        _sc_body,
        out_type=[
            jax.ShapeDtypeStruct((NW, NP), jnp.float32),
            jax.ShapeDtypeStruct((NC, NP, D), jnp.float32),
        ],
        mesh=mesh,
        scratch_types=[
            pltpu.VMEM((NP,), jnp.float32),
            pltpu.VMEM((NP,), jnp.float32),
            pltpu.VMEM((NP,), jnp.float32),
            pltpu.VMEM((SUBC, D), jnp.float32),
            pltpu.VMEM((SUBC,), jnp.int32),
            pltpu.VMEM((SUBC,), jnp.int32),
            pltpu.VMEM((16,), jnp.float32),
            pltpu.VMEM_SHARED((NP, D), jnp.float32),
        ],
        compiler_params=cp,
    )(s2, eidx, fea, bvec)

    out = pl.pallas_call(
        _post_body,
        grid=(GRID,),
        in_specs=[
            pl.BlockSpec((2, BLK, D), lambda i: (0, i, 0)),
            pl.BlockSpec((BLK, NW), lambda i: (i, 0)),
            pl.BlockSpec((BLK, D), lambda i: (i, 0)),
            pl.BlockSpec((1, D), lambda i: (0, 0)),
        ],
        out_specs=pl.BlockSpec((BLK, D), lambda i: (i, 0)),
        out_shape=jax.ShapeDtypeStruct((NP, D), jnp.float32),
    )(agg_p, jnp.transpose(den_p), xp, bo2)

    return (out[:N], bit_sum)


# speedup vs baseline: 11.6336x; 1.5081x over previous
import dataclasses

import jax
import jax.numpy as jnp
from jax import lax
from jax.experimental import pallas as pl
from jax.experimental.pallas import tpu as pltpu
from jax.experimental.pallas import tpu_sc as plsc

N = 10000
NP = 10240
E = 320000
D = 128
NC = 2
NS = 16
NW = NC * NS
E2 = 327680
EPT = E2 // NW
SUBC = 32
NSC = EPT // SUBC
NR = 4
NPT = NP // NS
BLK = 1024
GRID = NP // BLK


def _pre_body(x_ref, w_ref, b_ref, ai_ref, aj_ref, fea_ref, s2_ref, mx_ref):
    i = pl.program_id(0)
    fea = jnp.dot(x_ref[...], w_ref[...], preferred_element_type=jnp.float32,
                  precision=lax.Precision.HIGHEST) + b_ref[...]
    fea_ref[...] = fea
    dn = (((1,), (1,)), ((), ()))
    si = lax.dot_general(ai_ref[...], fea, dn, preferred_element_type=jnp.float32,
                         precision=lax.Precision.HIGHEST)
    sj = lax.dot_general(aj_ref[...], fea, dn, preferred_element_type=jnp.float32,
                         precision=lax.Precision.HIGHEST)
    s2_ref[...] = jnp.concatenate(
        [si, sj, jnp.zeros((6, si.shape[1]), jnp.float32)], axis=0)

    @pl.when(i == 0)
    def _():
        mx_ref[...] = jnp.full((8, 128), -1e30, jnp.float32)

    rows = lax.broadcasted_iota(jnp.int32, (8, 128), 0)
    mx_ref[...] = jnp.maximum(mx_ref[...],
                              jnp.where(rows < 4, jnp.max(si), jnp.max(sj)))


def _sc_body(s2_hbm, eidx_hbm, fea_hbm, bv_hbm, den_hbm, agg_hbm, *scr):
    si_v, sj_v, den_v = scr[0:3]
    rows = scr[3:7]
    dstb = scr[7:11]
    idxd = scr[11:15]
    idxs = scr[15:19]
    bv_v = scr[19]
    acc_s = scr[20]
    gsem = scr[21:25]
    ssem = scr[25:29]
    dsem = scr[29:33]
    xsem = scr[33:37]

    cid = lax.axis_index("c")
    sid = lax.axis_index("s")
    wid = cid * NS + sid

    pltpu.sync_copy(s2_hbm.at[0], si_v)
    pltpu.sync_copy(s2_hbm.at[1], sj_v)
    pltpu.sync_copy(bv_hbm, bv_v)
    bv = bv_v[...]

    z16 = jnp.zeros((16,), jnp.float32)

    @pl.loop(0, SUBC)
    def _(r):
        @pl.loop(0, D, step=16)
        def _(c):
            rows[0][r, pl.ds(c, 16)] = z16

    @pl.loop(0, NPT, step=SUBC)
    def _(r):
        pltpu.sync_copy(rows[0], acc_s.at[pl.ds(sid * NPT + r, SUBC)])

    @pl.loop(0, NP, step=16)
    def _(i):
        den_v[pl.ds(i, 16)] = z16

    plsc.subcore_barrier()

    def fetch_idx(ch, v):
        off = lax.rem(ch, NSC) * SUBC
        pltpu.async_copy(eidx_hbm.at[1, wid, pl.ds(off, SUBC)], idxd[v], dsem[v])
        pltpu.async_copy(eidx_hbm.at[0, wid, pl.ds(off, SUBC)], idxs[v], xsem[v])

    def wait_idx(v):
        pltpu.make_async_copy(eidx_hbm.at[1, wid, pl.ds(0, SUBC)], idxd[v], dsem[v]).wait()
        pltpu.make_async_copy(eidx_hbm.at[0, wid, pl.ds(0, SUBC)], idxs[v], xsem[v]).wait()

    def issue_gather(v):
        pltpu.async_copy(fea_hbm.at[idxs[v]], rows[v], gsem[v])

    def wait_gather(v):
        pltpu.make_async_copy(fea_hbm.at[idxs[v]], rows[v], gsem[v]).wait()

    def issue_scatter(v):
        pltpu.async_copy(rows[v], acc_s.at[dstb[v]], ssem[v], add=True)

    def wait_scatter(v):
        pltpu.make_async_copy(rows[v], acc_s.at[dstb[v]], ssem[v]).wait()

    for v in range(NR):
        fetch_idx(v, v)
    for v in range(2):
        wait_idx(v)
        issue_gather(v)

    @pl.loop(0, NSC, step=NR)
    def _(base):
        for u in range(NR):
            k = base + u
            u2 = (u + 2) % NR
            wait_gather(u)
            for g in range(0, SUBC, 16):
                d = idxd[u][pl.ds(g, 16)]
                s = idxs[u][pl.ds(g, 16)]
                a = plsc.load_gather(si_v, [d]) + plsc.load_gather(sj_v, [s])
                a = jnp.where(a > 0, a, 0.2 * a)
                ex = jnp.exp(a - bv)
                plsc.addupdate_scatter(den_v, [d], ex)
                dstb[u][pl.ds(g, 16)] = d
                for j in range(16):
                    e = ex[j]
                    for c in range(0, D, 16):
                        rows[u][g + j, pl.ds(c, 16)] = rows[u][g + j, pl.ds(c, 16)] * e
            issue_scatter(u)
            fetch_idx(k + NR, u)

            @pl.when(k >= 2)
            def _():
                wait_scatter(u2)

            wait_idx(u2)
            issue_gather(u2)

    for v in range(2):
        wait_gather(v)
    for v in range(2, NR):
        wait_scatter(v)
        wait_idx(v)

    pltpu.sync_copy(den_v, den_hbm.at[wid])

    plsc.subcore_barrier()

    pltpu.sync_copy(acc_s.at[pl.ds(sid * NPT, NPT)],
                    agg_hbm.at[cid, pl.ds(sid * NPT, NPT)])


def _post_body(ap_ref, dp_ref, x_ref, bo_ref, o_ref):
    agg = ap_ref[0] + ap_ref[1]
    den = jnp.sum(dp_ref[...], axis=1, keepdims=True)
    o_ref[...] = agg / (den + 1e-16) + x_ref[...] + bo_ref[...]


def kernel(x, edge_index, bit_sum, W, lin_b, att, b_out):
    xp = jnp.concatenate([x, jnp.zeros((NP - N, D), jnp.float32)], axis=0)
    ai = att[0, 0, :D].reshape(1, D)
    aj = att[0, 0, D:].reshape(1, D)
    lb2 = lin_b.reshape(1, D)
    bo2 = b_out.reshape(1, D)
    epad = jnp.full((2, E2 - E), N, dtype=jnp.int32)
    eidx = jnp.concatenate([edge_index, epad], axis=1).reshape(2, NW, EPT)

    fea, s2, mx = pl.pallas_call(
        _pre_body,
        grid=(GRID,),
        in_specs=[
            pl.BlockSpec((BLK, D), lambda i: (i, 0)),
            pl.BlockSpec((D, D), lambda i: (0, 0)),
            pl.BlockSpec((1, D), lambda i: (0, 0)),
            pl.BlockSpec((1, D), lambda i: (0, 0)),
            pl.BlockSpec((1, D), lambda i: (0, 0)),
        ],
        out_specs=[
            pl.BlockSpec((BLK, D), lambda i: (i, 0)),
            pl.BlockSpec((8, BLK), lambda i: (0, i)),
            pl.BlockSpec((8, 128), lambda i: (0, 0)),
        ],
        out_shape=[
            jax.ShapeDtypeStruct((NP, D), jnp.float32),
            jax.ShapeDtypeStruct((8, NP), jnp.float32),
            jax.ShapeDtypeStruct((8, 128), jnp.float32),
        ],
    )(xp, W, lb2, ai, aj)

    b0 = mx[0, 0] + mx[4, 0]
    bound = jnp.where(b0 > 0, b0, 0.2 * b0)
    bvec = jnp.full((16,), bound, jnp.float32)

    mesh = plsc.VectorSubcoreMesh(core_axis_name="c", subcore_axis_name="s")
    cp = pltpu.CompilerParams()
    if "needs_layout_passes" in pltpu.CompilerParams.__dataclass_fields__:
        cp = dataclasses.replace(cp, needs_layout_passes=False)
    den_p, agg_p = pl.kernel(
        _sc_body,
        out_type=[
            jax.ShapeDtypeStruct((NW, NP), jnp.float32),
            jax.ShapeDtypeStruct((NC, NP, D), jnp.float32),
        ],
        mesh=mesh,
        scratch_types=(
            [
                pltpu.VMEM((NP,), jnp.float32),
                pltpu.VMEM((NP,), jnp.float32),
                pltpu.VMEM((NP,), jnp.float32),
            ]
            + [pltpu.VMEM((SUBC, D), jnp.float32)] * NR
            + [pltpu.VMEM((SUBC,), jnp.int32)] * (3 * NR)
            + [
                pltpu.VMEM((16,), jnp.float32),
                pltpu.VMEM_SHARED((NP, D), jnp.float32),
            ]
            + [pltpu.SemaphoreType.DMA] * (4 * NR)
        ),
        compiler_params=cp,
    )(s2, eidx, fea, bvec)

    out = pl.pallas_call(
        _post_body,
        grid=(GRID,),
        in_specs=[
            pl.BlockSpec((2, BLK, D), lambda i: (0, i, 0)),
            pl.BlockSpec((BLK, NW), lambda i: (i, 0)),
            pl.BlockSpec((BLK, D), lambda i: (i, 0)),
            pl.BlockSpec((1, D), lambda i: (0, 0)),
        ],
        out_specs=pl.BlockSpec((BLK, D), lambda i: (i, 0)),
        out_shape=jax.ShapeDtypeStruct((NP, D), jnp.float32),
    )(agg_p, jnp.transpose(den_p), xp, bo2)

    return (out[:N], bit_sum)
